# baseline (speedup 1.0000x reference)
import functools

import jax
import jax.numpy as jnp
from jax import lax
from jax.experimental import pallas as pl
from jax.experimental.pallas import tpu as pltpu

N = 20000
C = 81
P = 128


def _tc_body(colt_ref, lab_ref, w_ref, pcl_ref, pcp_ref, imw_ref, iml_ref,
             out_ref, col_vmem, sem):
    cp = pltpu.make_async_copy(colt_ref.at[pl.ds(0, 1), :], col_vmem, sem)
    cp.start()
    cp.wait()
    col = col_vmem[...]
    lab = lab_ref[...].reshape(1, N)
    w = w_ref[...].reshape(1, N)
    bg_active = (iml_ref[0, 0] != 0.0).astype(jnp.float32)
    mask = (lab == 0).astype(jnp.float32)
    bg = -bg_active * jnp.sum(mask * w * jnp.log(col), axis=(0, 1),
                              keepdims=True)

    pcl = pcl_ref[...]
    iota_c = lax.broadcasted_iota(jnp.int32, (C, P), 0)
    onehot = (iota_c == pcl).astype(jnp.float32)
    gathered = lax.dot_general(
        iml_ref[...], onehot, (((1,), (0,)), ((), ())),
        preferred_element_type=jnp.float32)
    fg_active = (gathered != 0.0) & (pcl > 0)
    fg_vals = imw_ref[...] * jnp.log(pcp_ref[...])
    fg = -jnp.sum(jnp.where(fg_active, fg_vals, 0.0), axis=(0, 1),
                  keepdims=True)

    out_ref[...] = (bg + fg) * (1.0 / N)


@functools.partial(jax.jit, static_argnames=())
def kernel(pcl_prob, labels, cls_loss_weights, gt_assignment, pc_labels,
           pc_probs, pc_count, img_cls_loss_weights, im_labels_real):
    del gt_assignment, pc_count
    probt = pcl_prob.T
    out = pl.pallas_call(
        _tc_body,
        grid=(1,),
        in_specs=[
            pl.BlockSpec(memory_space=pltpu.MemorySpace.HBM),
            pl.BlockSpec((N,), lambda i: (0,)),
            pl.BlockSpec((N,), lambda i: (0,)),
            pl.BlockSpec((1, P), lambda i: (0, 0)),
            pl.BlockSpec((1, P), lambda i: (0, 0)),
            pl.BlockSpec((1, P), lambda i: (0, 0)),
            pl.BlockSpec((1, C), lambda i: (0, 0)),
        ],
        out_specs=pl.BlockSpec((1, 1), lambda i: (0, 0)),
        out_shape=jax.ShapeDtypeStruct((1, 1), jnp.float32),
        scratch_shapes=[
            pltpu.VMEM((1, N), jnp.float32),
            pltpu.SemaphoreType.DMA,
        ],
    )(probt, labels, cls_loss_weights,
      pc_labels.reshape(1, P), pc_probs.reshape(1, P),
      img_cls_loss_weights.reshape(1, P), im_labels_real.reshape(1, C))
    return out[0, 0]


# speedup vs baseline: 1.2986x; 1.2986x over previous
import functools

import jax
import jax.numpy as jnp
from jax import lax
from jax.experimental import pallas as pl
from jax.experimental.pallas import tpu as pltpu

N = 20000
C = 81
P = 128


def _tc_body(colt_ref, lab_ref, w_ref, pcl_ref, pcp_ref, imw_ref, iml_ref,
             out_ref):
    col = colt_ref[0:1, :]
    lab = lab_ref[...].reshape(1, N)
    w = w_ref[...].reshape(1, N)
    bg_active = (iml_ref[0, 0] != 0.0).astype(jnp.float32)
    mask = (lab == 0).astype(jnp.float32)
    bg = -bg_active * jnp.sum(mask * w * jnp.log(col), axis=(0, 1),
                              keepdims=True)

    pcl = pcl_ref[...]
    iota_c = lax.broadcasted_iota(jnp.int32, (C, P), 0)
    onehot = (iota_c == pcl).astype(jnp.float32)
    gathered = lax.dot_general(
        iml_ref[...], onehot, (((1,), (0,)), ((), ())),
        preferred_element_type=jnp.float32)
    fg_active = (gathered != 0.0) & (pcl > 0)
    fg_vals = imw_ref[...] * jnp.log(pcp_ref[...])
    fg = -jnp.sum(jnp.where(fg_active, fg_vals, 0.0), axis=(0, 1),
                  keepdims=True)

    out_ref[...] = (bg + fg) * (1.0 / N)


@functools.partial(jax.jit, static_argnames=())
def kernel(pcl_prob, labels, cls_loss_weights, gt_assignment, pc_labels,
           pc_probs, pc_count, img_cls_loss_weights, im_labels_real):
    del gt_assignment, pc_count
    probt = pcl_prob.T
    out = pl.pallas_call(
        _tc_body,
        grid=(1,),
        in_specs=[
            pl.BlockSpec((8, N), lambda i: (0, 0)),
            pl.BlockSpec((N,), lambda i: (0,)),
            pl.BlockSpec((N,), lambda i: (0,)),
            pl.BlockSpec((1, P), lambda i: (0, 0)),
            pl.BlockSpec((1, P), lambda i: (0, 0)),
            pl.BlockSpec((1, P), lambda i: (0, 0)),
            pl.BlockSpec((1, C), lambda i: (0, 0)),
        ],
        out_specs=pl.BlockSpec((1, 1), lambda i: (0, 0)),
        out_shape=jax.ShapeDtypeStruct((1, 1), jnp.float32),
    )(probt, labels, cls_loss_weights,
      pc_labels.reshape(1, P), pc_probs.reshape(1, P),
      img_cls_loss_weights.reshape(1, P), im_labels_real.reshape(1, C))
    return out[0, 0]
